# baseline (speedup 1.0000x reference)
import functools

import jax
import jax.numpy as jnp
from jax import lax
from jax.experimental import pallas as pl
from jax.experimental.pallas import tpu as pltpu
from jax.experimental.pallas import tpu_sc as plsc

EMBD = 300
GW = 128
NG = 3
DP = GW * NG
DH = 2 * DP
NL = 5
NN = 10000
NE = 160000
NF = 2000
ND = 4000
ITEMP = 25.0
CNTCOL = 301 - 2 * GW

EC = 80
IB = 31
CPT = (NE // EC) // 16
NROW_CH = NN // EC
NPOOL_CH = NF // EC

_MESH = plsc.VectorSubcoreMesh(core_axis_name="c", subcore_axis_name="s")


def _fill(ref, nrow, value):
    v = jnp.full((16,), value, dtype=ref.dtype)
    ncol = ref.shape[1] // 16

    def body(r, _):
        for j in range(ncol):
            ref[r, pl.ds(j * 16, 16)] = v
        return 0

    lax.fori_loop(0, nrow, body, 0)


def _mp_body(hg0, hg1, hg2, eg0, eg1, eg2, src, dst,
             agg0, agg1, agg2a, agg2b,
             agg_sp, sv, dv, dv2, hb, zb,
             esem0, esem1, esem2, gsem0, gsem1, gsem2,
             ssem0, ssem1, ssem2):
    c = lax.axis_index("c")
    t = lax.axis_index("s")
    esem = (esem0, esem1, esem2)
    gsem = (gsem0, gsem1, gsem2)
    ssem = (ssem0, ssem1, ssem2)

    def zero_acc():
        for k in range(8):
            i = t + k * 16

            @pl.when(i < NROW_CH)
            def _():
                for r in range(EC // 8):
                    pltpu.sync_copy(zb, agg_sp.at[pl.ds(i * EC + r * 8, 8)])

    def write_acc(out_ref):
        for k in range(8):
            i = t + k * 16

            @pl.when(i < NROW_CH)
            def _():
                rows = pl.ds(i * EC, EC)
                pltpu.sync_copy(agg_sp.at[rows], out_ref.at[rows])

    def run_batch(h_tab, e_tab, b0, n):
        assert (n - 1) % 3 == 0 and n >= 4
        base0 = (t * CPT + b0) * EC
        pltpu.sync_copy(src.at[pl.ds(base0, n * EC)], sv.at[pl.ds(0, n * EC)])
        pltpu.sync_copy(dst.at[pl.ds(base0, n * EC)], dv.at[pl.ds(0, n * EC)])

        def issue_e(k, m):
            pltpu.async_copy(e_tab.at[pl.ds(base0 + k * EC, EC)],
                             hb.at[m], esem[m])

        def wait_e(m):
            pltpu.make_async_copy(e_tab.at[pl.ds(0, EC)], hb.at[m],
                                  esem[m]).wait()

        def issue_g(k, m):
            pltpu.async_copy(h_tab.at[sv.at[pl.ds(k * EC, EC)]], hb.at[m],
                             gsem[m], add=True)

        def wait_g(m):
            pltpu.make_async_copy(h_tab.at[sv.at[pl.ds(0, EC)]], hb.at[m],
                                  gsem[m]).wait()

        def compute(m):
            def row(r, _):
                for u in range(2):
                    for j in range(GW // 16):
                        s = pl.ds(j * 16, 16)
                        hb[m, 2 * r + u, s] = jnp.maximum(
                            hb[m, 2 * r + u, s], 0.0)
                return 0

            lax.fori_loop(0, EC // 2, row, 0)

        def issue_s(k, m):
            for j in range(EC // 16):
                dv2[m, pl.ds(j * 16, 16)] = dv[pl.ds(k * EC + j * 16, 16)]
            pltpu.async_copy(hb.at[m], agg_sp.at[dv2.at[m]], ssem[m],
                             add=True)

        def wait_s(m):
            pltpu.make_async_copy(hb.at[m], agg_sp.at[dv2.at[m]],
                                  ssem[m]).wait()

        def body(k, m, first):
            m1 = (m + 1) % 3
            m2 = (m + 2) % 3

            def pg():
                wait_e(m1)
                issue_g(k + 1, m1)

            def pe():
                if not first:
                    wait_s(m2)
                issue_e(k + 2, m2)

            if first:
                pg()
                pe()
            else:
                pl.when(k + 1 < n)(pg)
                pl.when(k + 2 < n)(pe)
            wait_g(m)
            compute(m)
            issue_s(k, m)

        pltpu.sync_copy(e_tab.at[pl.ds(base0, EC)], hb.at[0])
        issue_e(1, 1)
        issue_g(0, 0)
        body(0, 0, True)

        def triple(j, _):
            k = 1 + 3 * j
            body(k, 1, False)
            body(k + 1, 2, False)
            body(k + 2, 0, False)
            return 0

        lax.fori_loop(0, (n - 1) // 3, triple, 0)
        wait_s((n - 3) % 3)
        wait_s((n - 2) % 3)
        wait_s((n - 1) % 3)

    def run(h_tab, e_tab, k0, nbatch, bsize):
        def bat(jb, _):
            run_batch(h_tab, e_tab, k0 + jb * bsize, bsize)
            return 0

        lax.fori_loop(0, nbatch, bat, 0)

    _fill(zb, 8, 0.0)
    zero_acc()
    plsc.subcore_barrier()

    @pl.when(c == 0)
    def _():
        run(hg0, eg0, 0, 5, 25)

    @pl.when(c == 1)
    def _():
        run(hg1, eg1, 0, 5, 25)

    plsc.subcore_barrier()

    @pl.when(c == 0)
    def _():
        write_acc(agg0)

    @pl.when(c == 1)
    def _():
        write_acc(agg1)

    plsc.subcore_barrier()

    zero_acc()
    plsc.subcore_barrier()

    @pl.when(c == 0)
    def _():
        run(hg2, eg2, 0, 2, 31)

    @pl.when(c == 1)
    def _():
        run_batch(hg2, eg2, 62, 31)
        run(hg2, eg2, 93, 2, 16)

    plsc.subcore_barrier()

    @pl.when(c == 0)
    def _():
        write_acc(agg2a)

    @pl.when(c == 1)
    def _():
        write_acc(agg2b)


_mp = pl.kernel(
    _mp_body,
    out_type=tuple(
        jax.ShapeDtypeStruct((NN, GW), jnp.float32) for _ in range(4)),
    mesh=_MESH,
    scratch_types=[
        pltpu.VMEM_SHARED((NN, GW), jnp.float32),
        pltpu.VMEM((IB * EC,), jnp.int32),
        pltpu.VMEM((IB * EC,), jnp.int32),
        pltpu.VMEM((3, EC), jnp.int32),
        pltpu.VMEM((3, EC, GW), jnp.float32),
        pltpu.VMEM((8, GW), jnp.float32),
        pltpu.SemaphoreType.DMA,
        pltpu.SemaphoreType.DMA,
        pltpu.SemaphoreType.DMA,
        pltpu.SemaphoreType.DMA,
        pltpu.SemaphoreType.DMA,
        pltpu.SemaphoreType.DMA,
        pltpu.SemaphoreType.DMA,
        pltpu.SemaphoreType.DMA,
        pltpu.SemaphoreType.DMA,
    ],
)


def _pool_body(hg0, hg1, hg2, fb, p0, p1, p2a, p2b,
               pool_sp, fb_v, hrow_v):
    c = lax.axis_index("c")
    t = lax.axis_index("s")

    def zero_pool():
        for k in range(2):
            i = t + k * 16

            @pl.when(i < NPOOL_CH)
            def _():
                pltpu.sync_copy(hrow_v, pool_sp.at[pl.ds(i * EC, EC)])

    def pool_step(base, h_tab):
        pltpu.sync_copy(fb.at[pl.ds(base, EC)], fb_v)
        pltpu.sync_copy(h_tab.at[pl.ds(base, EC)], hrow_v)
        pltpu.sync_copy(hrow_v, pool_sp.at[fb_v], add=True)

    def write_pool(out_ref):
        for k in range(2):
            i = t + k * 16

            @pl.when(i < NPOOL_CH)
            def _():
                rows = pl.ds(i * EC, EC)
                pltpu.sync_copy(pool_sp.at[rows], out_ref.at[rows])

    _fill(hrow_v, EC, 0.0)
    zero_pool()
    plsc.subcore_barrier()
    for k in range(8):
        i = t + k * 16

        @pl.when(i < NROW_CH)
        def _():
            base = i * EC

            @pl.when(c == 0)
            def _():
                pool_step(base, hg0)

            @pl.when(c == 1)
            def _():
                pool_step(base, hg1)

    plsc.subcore_barrier()

    @pl.when(c == 0)
    def _():
        write_pool(p0)

    @pl.when(c == 1)
    def _():
        write_pool(p1)

    plsc.subcore_barrier()

    _fill(hrow_v, EC, 0.0)
    zero_pool()
    plsc.subcore_barrier()
    for k in range(8):
        i = t + k * 16

        @pl.when(jnp.logical_and(i < NROW_CH, lax.rem(i, 2) == c))
        def _():
            pool_step(i * EC, hg2)

    plsc.subcore_barrier()

    @pl.when(c == 0)
    def _():
        write_pool(p2a)

    @pl.when(c == 1)
    def _():
        write_pool(p2b)


_pool = pl.kernel(
    _pool_body,
    out_type=tuple(
        jax.ShapeDtypeStruct((NF, GW), jnp.float32) for _ in range(4)),
    mesh=_MESH,
    scratch_types=[
        pltpu.VMEM_SHARED((NF, GW), jnp.float32),
        pltpu.VMEM((EC,), jnp.int32),
        pltpu.VMEM((EC, GW), jnp.float32),
    ],
)


def _egen_body(ea_ref, we_ref, be_ref, *outs):
    ea = ea_ref[...]
    for l in range(NL):
        e = (jnp.dot(ea, we_ref[l], preferred_element_type=jnp.float32)
             + be_ref[l])
        for g in range(NG):
            outs[l * NG + g][...] = e[:, g * GW:(g + 1) * GW]


def _egen(edge_attr, Wep, bep):
    blk = 1000
    grid = NE // blk
    return pl.pallas_call(
        _egen_body,
        grid=(grid,),
        in_specs=[
            pl.BlockSpec((blk, 16), lambda i: (i, 0)),
            pl.BlockSpec((NL, 16, DP), lambda i: (0, 0, 0)),
            pl.BlockSpec((NL, 1, DP), lambda i: (0, 0, 0)),
        ],
        out_specs=[pl.BlockSpec((blk, GW), lambda i: (i, 0))] * (NL * NG),
        out_shape=[jax.ShapeDtypeStruct((NE, GW), jnp.float32)] * (NL * NG),
    )(edge_attr, Wep, bep)


def _mlp_body(hg0, hg1, hg2, a0, a1, a2a, a2b,
              w1_ref, b1_ref, w2_ref, b2_ref, o0, o1, o2, *, final):
    h = jnp.concatenate([hg0[...], hg1[...], hg2[...]], axis=1)
    agg = jnp.concatenate(
        [a0[...], a1[...], a2a[...] + a2b[...]], axis=1)
    z = h + agg
    mid = jnp.maximum(
        jnp.dot(z, w1_ref[...], preferred_element_type=jnp.float32)
        + b1_ref[...], 0.0)
    out = (jnp.dot(mid, w2_ref[...], preferred_element_type=jnp.float32)
           + b2_ref[...])
    if not final:
        out = jnp.maximum(out, 0.0)
    o0[...] = out[:, :GW]
    o1[...] = out[:, GW:2 * GW]
    og2 = out[:, 2 * GW:]
    if final:
        lane = lax.broadcasted_iota(jnp.int32, og2.shape, 1)
        og2 = jnp.where(lane == CNTCOL, 1.0, og2)
    o2[...] = og2


def _mlp(hg, ag, W1l, b1l, W2l, b2l, final):
    blk = 1000
    grid = NN // blk
    gb = lambda: pl.BlockSpec((blk, GW), lambda i: (i, 0))
    return pl.pallas_call(
        functools.partial(_mlp_body, final=final),
        grid=(grid,),
        in_specs=[
            gb(), gb(), gb(), gb(), gb(), gb(), gb(),
            pl.BlockSpec((DP, DH), lambda i: (0, 0)),
            pl.BlockSpec((1, DH), lambda i: (0, 0)),
            pl.BlockSpec((DH, DP), lambda i: (0, 0)),
            pl.BlockSpec((1, DP), lambda i: (0, 0)),
        ],
        out_specs=[gb(), gb(), gb()],
        out_shape=[jax.ShapeDtypeStruct((NN, GW), jnp.float32)] * NG,
    )(*hg, *ag, W1l, b1l, W2l, b2l)


def _head_body(p0, p1, p2a, p2b, pw1, pb1, pw2, pb2, qw1, qb1, qw2, qb2,
               f0_ref, f1_ref):
    g2 = p2a[...] + p2b[...]
    pooled = jnp.concatenate([p0[...], p1[...], g2], axis=1)
    cnt = jnp.maximum(g2[:, CNTCOL:CNTCOL + 1], 1.0)
    mean = pooled / cnt
    out = (jnp.dot(
        jnp.maximum(jnp.dot(mean, pw1[...],
                            preferred_element_type=jnp.float32) + pb1[...],
                    0.0),
        pw2[...], preferred_element_type=jnp.float32) + pb2[...])
    n0 = jnp.sqrt(jnp.sum(out * out, axis=1, keepdims=True))
    f0_ref[...] = out / jnp.maximum(n0, 1e-12)
    out2 = (jnp.dot(
        jnp.maximum(jnp.dot(out, qw1[...],
                            preferred_element_type=jnp.float32) + qb1[...],
                    0.0),
        qw2[...], preferred_element_type=jnp.float32) + qb2[...])
    n1 = jnp.sqrt(jnp.sum(out2 * out2, axis=1, keepdims=True))
    f1_ref[...] = out2 / jnp.maximum(n1, 1e-12)


def _head(pools, PW1p, Pb1p, PW2p, Pb2p, QW1p, Qb1p, QW2p, Qb2p):
    return pl.pallas_call(
        _head_body,
        out_shape=[jax.ShapeDtypeStruct((NF, DP), jnp.float32)] * 2,
    )(*pools, PW1p, Pb1p, PW2p, Pb2p, QW1p, Qb1p, QW2p, Qb2p)


def _logits_body(f0_ref, f1_ref, out_ref):
    out_ref[...] = ITEMP * lax.dot_general(
        f0_ref[...], f1_ref[...], (((1,), (1,)), ((), ())),
        preferred_element_type=jnp.float32)


def _logits(f0, f1):
    blk = 400
    return pl.pallas_call(
        _logits_body,
        grid=(NF // blk,),
        in_specs=[
            pl.BlockSpec((blk, DP), lambda i: (i, 0)),
            pl.BlockSpec((NF, DP), lambda i: (0, 0)),
        ],
        out_specs=pl.BlockSpec((blk, NF), lambda i: (i, 0)),
        out_shape=jax.ShapeDtypeStruct((NF, NF), jnp.float32),
    )(f0, f1)


def _labels_body(d0_ref, d1_ref, out_ref):
    i = pl.program_id(0)
    blk = out_ref.shape[0]
    ii = lax.broadcasted_iota(jnp.int32, (ND, blk), 1)
    jj = lax.broadcasted_iota(jnp.int32, (ND, NF), 1)
    a = (d0_ref[...] == ii + i * blk).astype(jnp.bfloat16)
    b = (d1_ref[...] == jj).astype(jnp.bfloat16)
    out_ref[...] = lax.dot_general(
        a, b, (((0,), (0,)), ((), ())), preferred_element_type=jnp.float32)


def _labels(d0, d1):
    blk = 400
    return pl.pallas_call(
        _labels_body,
        grid=(NF // blk,),
        in_specs=[
            pl.BlockSpec((ND, 1), lambda i: (0, 0)),
            pl.BlockSpec((ND, 1), lambda i: (0, 0)),
        ],
        out_specs=pl.BlockSpec((blk, NF), lambda i: (i, 0)),
        out_shape=jax.ShapeDtypeStruct((NF, NF), jnp.float32),
    )(d0, d1)


def kernel(x, edge_index, edge_attr, frag_batch, dangling_edge_index,
           We, be, W1, b1, W2, b2, PW1, Pb1, PW2, Pb2, QW1, Qb1, QW2, Qb2):
    f32 = jnp.float32

    def pad(a, tgt):
        return jnp.pad(a, [(0, t - s) for s, t in zip(a.shape, tgt)])

    Wep = pad(We.astype(f32), (NL, 16, DP))
    bep = pad(be.astype(f32), (NL, DP)).reshape(NL, 1, DP)
    W1p = pad(W1.astype(f32), (NL, DP, DH))
    b1p = pad(b1.astype(f32), (NL, DH)).reshape(NL, 1, DH)
    W2p = pad(W2.astype(f32), (NL, DH, DP))
    b2p = pad(b2.astype(f32), (NL, DP)).reshape(NL, 1, DP)
    PW1p = pad(PW1.astype(f32), (DP, DP))
    Pb1p = pad(Pb1.astype(f32), (DP,)).reshape(1, DP)
    PW2p = pad(PW2.astype(f32), (DP, DP))
    Pb2p = pad(Pb2.astype(f32), (DP,)).reshape(1, DP)
    QW1p = pad(QW1.astype(f32), (DP, DP))
    Qb1p = pad(Qb1.astype(f32), (DP,)).reshape(1, DP)
    QW2p = pad(QW2.astype(f32), (DP, DP))
    Qb2p = pad(Qb2.astype(f32), (DP,)).reshape(1, DP)

    src = edge_index[0].astype(jnp.int32)
    dst = edge_index[1].astype(jnp.int32)
    fb = frag_batch.astype(jnp.int32)
    d0 = dangling_edge_index[0].astype(jnp.int32).reshape(ND, 1)
    d1 = dangling_edge_index[1].astype(jnp.int32).reshape(ND, 1)

    xp = pad(x.astype(f32), (NN, DP))
    hg = [xp[:, g * GW:(g + 1) * GW] for g in range(NG)]
    e_list = _egen(edge_attr.astype(f32), Wep, bep)
    for l in range(NL):
        eg = e_list[l * NG:(l + 1) * NG]
        ag = _mp(*hg, *eg, src, dst)
        hg = _mlp(hg, ag, W1p[l], b1p[l], W2p[l], b2p[l],
                  final=(l == NL - 1))

    pools = _pool(*hg, fb)
    f0, f1 = _head(pools, PW1p, Pb1p, PW2p, Pb2p, QW1p, Qb1p, QW2p, Qb2p)
    logits = _logits(f0, f1)
    labels = _labels(d0, d1)
    return (logits, labels)


# speedup vs baseline: 1.0793x; 1.0793x over previous
import functools

import jax
import jax.numpy as jnp
from jax import lax
from jax.experimental import pallas as pl
from jax.experimental.pallas import tpu as pltpu
from jax.experimental.pallas import tpu_sc as plsc

EMBD = 300
GW = 128
NG = 3
DP = GW * NG
DH = 2 * DP
NL = 5
NN = 10000
NE = 160000
NF = 2000
ND = 4000
ITEMP = 25.0
CNTCOL = 301 - 2 * GW

EC = 80
IB = 31
CPT = (NE // EC) // 16
NROW_CH = NN // EC
NPOOL_CH = NF // EC

_MESH = plsc.VectorSubcoreMesh(core_axis_name="c", subcore_axis_name="s")


def _bdot(a, b):
    return jnp.dot(a.astype(jnp.bfloat16), b.astype(jnp.bfloat16),
                   preferred_element_type=jnp.float32)


def _fill(ref, nrow, value):
    v = jnp.full((16,), value, dtype=ref.dtype)
    ncol = ref.shape[1] // 16

    def body(r, _):
        for j in range(ncol):
            ref[r, pl.ds(j * 16, 16)] = v
        return 0

    lax.fori_loop(0, nrow, body, 0)


def _mp_body(hg0, hg1, hg2, eg0, eg1, eg2, src, dst,
             agg0, agg1, agg2a, agg2b,
             agg_sp, sv, dv, dv2, hb, eb, zb,
             dsem0, dsem1, ssem0, ssem1):
    c = lax.axis_index("c")
    t = lax.axis_index("s")
    dsem = (dsem0, dsem1)
    ssem = (ssem0, ssem1)

    def zero_acc():
        for k in range(8):
            i = t + k * 16

            @pl.when(i < NROW_CH)
            def _():
                for r in range(EC // 8):
                    pltpu.sync_copy(zb, agg_sp.at[pl.ds(i * EC + r * 8, 8)])

    def write_acc(out_ref):
        for k in range(8):
            i = t + k * 16

            @pl.when(i < NROW_CH)
            def _():
                rows = pl.ds(i * EC, EC)
                pltpu.sync_copy(agg_sp.at[rows], out_ref.at[rows])

    def run_batch(h_tab, e_tab, b0, n):
        base0 = (t * CPT + b0) * EC
        pltpu.sync_copy(src.at[pl.ds(base0, n * EC)], sv.at[pl.ds(0, n * EC)])
        pltpu.sync_copy(dst.at[pl.ds(base0, n * EC)], dv.at[pl.ds(0, n * EC)])

        def issue_dat(k, b):
            pltpu.async_copy(h_tab.at[sv.at[pl.ds(k * EC, EC)]], hb.at[b],
                             dsem[b])
            pltpu.async_copy(e_tab.at[pl.ds(base0 + k * EC, EC)],
                             eb.at[b], dsem[b])

        def wait_dat(b):
            pltpu.make_async_copy(h_tab.at[sv.at[pl.ds(0, EC)]], hb.at[b],
                                  dsem[b]).wait()
            pltpu.make_async_copy(e_tab.at[pl.ds(0, EC)], eb.at[b],
                                  dsem[b]).wait()

        def compute(b):
            def row(r, _):
                for u in range(2):
                    for j in range(GW // 16):
                        s = pl.ds(j * 16, 16)
                        hb[b, 2 * r + u, s] = jnp.maximum(
                            hb[b, 2 * r + u, s] + eb[b, 2 * r + u, s], 0.0)
                return 0

            lax.fori_loop(0, EC // 2, row, 0)

        def issue_scat(k, b):
            for j in range(EC // 16):
                dv2[b, pl.ds(j * 16, 16)] = dv[pl.ds(k * EC + j * 16, 16)]
            pltpu.async_copy(hb.at[b], agg_sp.at[dv2.at[b]], ssem[b],
                             add=True)

        def wait_scat(b):
            pltpu.make_async_copy(hb.at[b], agg_sp.at[dv2.at[b]],
                                  ssem[b]).wait()

        def step(k, b, prefetch):
            nb = 1 - b
            wait_scat(nb)
            if prefetch:
                @pl.when(k + 1 < n)
                def _():
                    issue_dat(k + 1, nb)

            wait_dat(b)
            compute(b)
            issue_scat(k, b)

        issue_dat(0, 0)
        issue_dat(1, 1)
        wait_dat(0)
        compute(0)
        issue_scat(0, 0)

        npairs = (n - 1) // 2

        def pair(j, _):
            step(2 * j + 1, 1, True)
            step(2 * j + 2, 0, True)
            return 0

        lax.fori_loop(0, npairs, pair, 0)
        if (n - 1) % 2 == 1:
            step(n - 1, (n - 1) % 2, False)
        wait_scat((n - 1) % 2)

    def run(h_tab, e_tab, k0, nbatch, bsize):
        def bat(jb, _):
            run_batch(h_tab, e_tab, k0 + jb * bsize, bsize)
            return 0

        lax.fori_loop(0, nbatch, bat, 0)

    _fill(zb, 8, 0.0)
    zero_acc()
    plsc.subcore_barrier()

    @pl.when(c == 0)
    def _():
        run(hg0, eg0, 0, 5, 25)

    @pl.when(c == 1)
    def _():
        run(hg1, eg1, 0, 5, 25)

    plsc.subcore_barrier()

    @pl.when(c == 0)
    def _():
        write_acc(agg0)

    @pl.when(c == 1)
    def _():
        write_acc(agg1)

    plsc.subcore_barrier()

    zero_acc()
    plsc.subcore_barrier()

    @pl.when(c == 0)
    def _():
        run(hg2, eg2, 0, 2, 31)

    @pl.when(c == 1)
    def _():
        run(hg2, eg2, 62, 3, 21)

    plsc.subcore_barrier()

    @pl.when(c == 0)
    def _():
        write_acc(agg2a)

    @pl.when(c == 1)
    def _():
        write_acc(agg2b)


_mp = pl.kernel(
    _mp_body,
    out_type=tuple(
        jax.ShapeDtypeStruct((NN, GW), jnp.float32) for _ in range(4)),
    mesh=_MESH,
    scratch_types=[
        pltpu.VMEM_SHARED((NN, GW), jnp.float32),
        pltpu.VMEM((IB * EC,), jnp.int32),
        pltpu.VMEM((IB * EC,), jnp.int32),
        pltpu.VMEM((2, EC), jnp.int32),
        pltpu.VMEM((2, EC, GW), jnp.float32),
        pltpu.VMEM((2, EC, GW), jnp.float32),
        pltpu.VMEM((8, GW), jnp.float32),
        pltpu.SemaphoreType.DMA,
        pltpu.SemaphoreType.DMA,
        pltpu.SemaphoreType.DMA,
        pltpu.SemaphoreType.DMA,
    ],
)


def _pool_body(hg0, hg1, hg2, fb, p0, p1, p2a, p2b,
               pool_sp, fb_v, hrow_v):
    c = lax.axis_index("c")
    t = lax.axis_index("s")

    def zero_pool():
        for k in range(2):
            i = t + k * 16

            @pl.when(i < NPOOL_CH)
            def _():
                pltpu.sync_copy(hrow_v, pool_sp.at[pl.ds(i * EC, EC)])

    def pool_step(base, h_tab):
        pltpu.sync_copy(fb.at[pl.ds(base, EC)], fb_v)
        pltpu.sync_copy(h_tab.at[pl.ds(base, EC)], hrow_v)
        pltpu.sync_copy(hrow_v, pool_sp.at[fb_v], add=True)

    def write_pool(out_ref):
        for k in range(2):
            i = t + k * 16

            @pl.when(i < NPOOL_CH)
            def _():
                rows = pl.ds(i * EC, EC)
                pltpu.sync_copy(pool_sp.at[rows], out_ref.at[rows])

    _fill(hrow_v, EC, 0.0)
    zero_pool()
    plsc.subcore_barrier()
    for k in range(8):
        i = t + k * 16

        @pl.when(i < NROW_CH)
        def _():
            base = i * EC

            @pl.when(c == 0)
            def _():
                pool_step(base, hg0)

            @pl.when(c == 1)
            def _():
                pool_step(base, hg1)

    plsc.subcore_barrier()

    @pl.when(c == 0)
    def _():
        write_pool(p0)

    @pl.when(c == 1)
    def _():
        write_pool(p1)

    plsc.subcore_barrier()

    _fill(hrow_v, EC, 0.0)
    zero_pool()
    plsc.subcore_barrier()
    for k in range(8):
        i = t + k * 16

        @pl.when(jnp.logical_and(i < NROW_CH, lax.rem(i, 2) == c))
        def _():
            pool_step(i * EC, hg2)

    plsc.subcore_barrier()

    @pl.when(c == 0)
    def _():
        write_pool(p2a)

    @pl.when(c == 1)
    def _():
        write_pool(p2b)


_pool = pl.kernel(
    _pool_body,
    out_type=tuple(
        jax.ShapeDtypeStruct((NF, GW), jnp.float32) for _ in range(4)),
    mesh=_MESH,
    scratch_types=[
        pltpu.VMEM_SHARED((NF, GW), jnp.float32),
        pltpu.VMEM((EC,), jnp.int32),
        pltpu.VMEM((EC, GW), jnp.float32),
    ],
)


def _egen_body(ea_ref, we_ref, be_ref, o0, o1, o2):
    ea = ea_ref[...]
    e = _bdot(ea, we_ref[...]) + be_ref[...]
    for g, o in enumerate((o0, o1, o2)):
        o[...] = e[:, g * GW:(g + 1) * GW]


def _egen(edge_attr, Wel, bel):
    blk = 1000
    grid = NE // blk
    return pl.pallas_call(
        _egen_body,
        grid=(grid,),
        in_specs=[
            pl.BlockSpec((blk, 16), lambda i: (i, 0)),
            pl.BlockSpec((16, DP), lambda i: (0, 0)),
            pl.BlockSpec((1, DP), lambda i: (0, 0)),
        ],
        out_specs=[pl.BlockSpec((blk, GW), lambda i: (i, 0))] * NG,
        out_shape=[jax.ShapeDtypeStruct((NE, GW), jnp.float32)] * NG,
    )(edge_attr, Wel, bel)


def _mlp_body(hg0, hg1, hg2, a0, a1, a2a, a2b,
              w1_ref, b1_ref, w2_ref, b2_ref, o0, o1, o2, *, final):
    h = jnp.concatenate([hg0[...], hg1[...], hg2[...]], axis=1)
    agg = jnp.concatenate(
        [a0[...], a1[...], a2a[...] + a2b[...]], axis=1)
    z = h + agg
    mid = jnp.maximum(_bdot(z, w1_ref[...]) + b1_ref[...], 0.0)
    out = _bdot(mid, w2_ref[...]) + b2_ref[...]
    if not final:
        out = jnp.maximum(out, 0.0)
    o0[...] = out[:, :GW]
    o1[...] = out[:, GW:2 * GW]
    og2 = out[:, 2 * GW:]
    if final:
        lane = lax.broadcasted_iota(jnp.int32, og2.shape, 1)
        og2 = jnp.where(lane == CNTCOL, 1.0, og2)
    o2[...] = og2


def _mlp(hg, ag, W1l, b1l, W2l, b2l, final):
    blk = 1000
    grid = NN // blk
    gb = lambda: pl.BlockSpec((blk, GW), lambda i: (i, 0))
    return pl.pallas_call(
        functools.partial(_mlp_body, final=final),
        grid=(grid,),
        in_specs=[
            gb(), gb(), gb(), gb(), gb(), gb(), gb(),
            pl.BlockSpec((DP, DH), lambda i: (0, 0)),
            pl.BlockSpec((1, DH), lambda i: (0, 0)),
            pl.BlockSpec((DH, DP), lambda i: (0, 0)),
            pl.BlockSpec((1, DP), lambda i: (0, 0)),
        ],
        out_specs=[gb(), gb(), gb()],
        out_shape=[jax.ShapeDtypeStruct((NN, GW), jnp.float32)] * NG,
    )(*hg, *ag, W1l, b1l, W2l, b2l)


def _head_body(p0, p1, p2a, p2b, pw1, pb1, pw2, pb2, qw1, qb1, qw2, qb2,
               f0_ref, f1_ref):
    g2 = p2a[...] + p2b[...]
    pooled = jnp.concatenate([p0[...], p1[...], g2], axis=1)
    cnt = jnp.maximum(g2[:, CNTCOL:CNTCOL + 1], 1.0)
    mean = pooled / cnt
    out = (_bdot(
        jnp.maximum(_bdot(mean, pw1[...]) + pb1[...], 0.0),
        pw2[...]) + pb2[...])
    n0 = jnp.sqrt(jnp.sum(out * out, axis=1, keepdims=True))
    f0_ref[...] = out / jnp.maximum(n0, 1e-12)
    out2 = (_bdot(
        jnp.maximum(_bdot(out, qw1[...]) + qb1[...], 0.0),
        qw2[...]) + qb2[...])
    n1 = jnp.sqrt(jnp.sum(out2 * out2, axis=1, keepdims=True))
    f1_ref[...] = out2 / jnp.maximum(n1, 1e-12)


def _head(pools, PW1p, Pb1p, PW2p, Pb2p, QW1p, Qb1p, QW2p, Qb2p):
    return pl.pallas_call(
        _head_body,
        out_shape=[jax.ShapeDtypeStruct((NF, DP), jnp.float32)] * 2,
    )(*pools, PW1p, Pb1p, PW2p, Pb2p, QW1p, Qb1p, QW2p, Qb2p)


def _logits_body(f0_ref, f1_ref, out_ref):
    out_ref[...] = ITEMP * lax.dot_general(
        f0_ref[...].astype(jnp.bfloat16), f1_ref[...].astype(jnp.bfloat16),
        (((1,), (1,)), ((), ())), preferred_element_type=jnp.float32)


def _logits(f0, f1):
    blk = 400
    return pl.pallas_call(
        _logits_body,
        grid=(NF // blk,),
        in_specs=[
            pl.BlockSpec((blk, DP), lambda i: (i, 0)),
            pl.BlockSpec((NF, DP), lambda i: (0, 0)),
        ],
        out_specs=pl.BlockSpec((blk, NF), lambda i: (i, 0)),
        out_shape=jax.ShapeDtypeStruct((NF, NF), jnp.float32),
    )(f0, f1)


def _labels_body(d0_ref, d1_ref, out_ref):
    i = pl.program_id(0)
    blk = out_ref.shape[0]
    ii = lax.broadcasted_iota(jnp.int32, (ND, blk), 1)
    jj = lax.broadcasted_iota(jnp.int32, (ND, NF), 1)
    a = (d0_ref[...] == ii + i * blk).astype(jnp.bfloat16)
    b = (d1_ref[...] == jj).astype(jnp.bfloat16)
    out_ref[...] = lax.dot_general(
        a, b, (((0,), (0,)), ((), ())), preferred_element_type=jnp.float32)


def _labels(d0, d1):
    blk = 400
    return pl.pallas_call(
        _labels_body,
        grid=(NF // blk,),
        in_specs=[
            pl.BlockSpec((ND, 1), lambda i: (0, 0)),
            pl.BlockSpec((ND, 1), lambda i: (0, 0)),
        ],
        out_specs=pl.BlockSpec((blk, NF), lambda i: (i, 0)),
        out_shape=jax.ShapeDtypeStruct((NF, NF), jnp.float32),
    )(d0, d1)


def kernel(x, edge_index, edge_attr, frag_batch, dangling_edge_index,
           We, be, W1, b1, W2, b2, PW1, Pb1, PW2, Pb2, QW1, Qb1, QW2, Qb2):
    f32 = jnp.float32

    def pad(a, tgt):
        return jnp.pad(a, [(0, t - s) for s, t in zip(a.shape, tgt)])

    Wep = pad(We.astype(f32), (NL, 16, DP))
    bep = pad(be.astype(f32), (NL, DP)).reshape(NL, 1, DP)
    W1p = pad(W1.astype(f32), (NL, DP, DH))
    b1p = pad(b1.astype(f32), (NL, DH)).reshape(NL, 1, DH)
    W2p = pad(W2.astype(f32), (NL, DH, DP))
    b2p = pad(b2.astype(f32), (NL, DP)).reshape(NL, 1, DP)
    PW1p = pad(PW1.astype(f32), (DP, DP))
    Pb1p = pad(Pb1.astype(f32), (DP,)).reshape(1, DP)
    PW2p = pad(PW2.astype(f32), (DP, DP))
    Pb2p = pad(Pb2.astype(f32), (DP,)).reshape(1, DP)
    QW1p = pad(QW1.astype(f32), (DP, DP))
    Qb1p = pad(Qb1.astype(f32), (DP,)).reshape(1, DP)
    QW2p = pad(QW2.astype(f32), (DP, DP))
    Qb2p = pad(Qb2.astype(f32), (DP,)).reshape(1, DP)

    src = edge_index[0].astype(jnp.int32)
    dst = edge_index[1].astype(jnp.int32)
    fb = frag_batch.astype(jnp.int32)
    d0 = dangling_edge_index[0].astype(jnp.int32).reshape(ND, 1)
    d1 = dangling_edge_index[1].astype(jnp.int32).reshape(ND, 1)

    xp = pad(x.astype(f32), (NN, DP))
    hg = [xp[:, g * GW:(g + 1) * GW] for g in range(NG)]
    ea32 = edge_attr.astype(f32)
    for l in range(NL):
        eg = _egen(ea32, Wep[l], bep[l])
        ag = _mp(*hg, *eg, src, dst)
        hg = _mlp(hg, ag, W1p[l], b1p[l], W2p[l], b2p[l],
                  final=(l == NL - 1))

    pools = _pool(*hg, fb)
    f0, f1 = _head(pools, PW1p, Pb1p, PW2p, Pb2p, QW1p, Qb1p, QW2p, Qb2p)
    logits = _logits(f0, f1)
    labels = _labels(d0, d1)
    return (logits, labels)
